# Initial kernel scaffold; baseline (speedup 1.0000x reference)
import jax
import jax.numpy as jnp
from jax.experimental import pallas as pl


def kernel(dist_mat, table):
    raise NotImplementedError("write your pallas kernel here")


# speedup vs baseline: 3.5977x; 3.5977x over previous
import functools

import jax
import jax.numpy as jnp
from jax import lax
from jax.experimental import pallas as pl
from jax.experimental.pallas import tpu as pltpu
from jax.experimental.pallas import tpu_sc as plsc

SEQ = 2048
HIDDEN = 64
LANES = 2 * HIDDEN
B = SEQ * SEQ
ROW = 128
NROWS = B // ROW
NW = 32
ROWS_PER_W = NROWS // NW


def _make_gather():
    mesh = plsc.VectorSubcoreMesh(core_axis_name="c", subcore_axis_name="s")

    @functools.partial(
        pl.kernel,
        mesh=mesh,
        out_type=jax.ShapeDtypeStruct((B, HIDDEN), jnp.float32),
        scratch_types=[
            pltpu.VMEM((ROW,), jnp.int32),
            pltpu.VMEM((ROW, HIDDEN), jnp.float32),
            pltpu.SemaphoreType.DMA,
        ],
        compiler_params=pltpu.CompilerParams(use_tc_tiling_on_sc=False),
    )
    def gather_kernel(table_hbm, idx_hbm, out_hbm, idx_v, rows_v, sem):
        c = lax.axis_index("c")
        s = lax.axis_index("s")
        wid = s * 2 + c

        def step(g, carry):
            row = wid * ROWS_PER_W + g
            pltpu.sync_copy(idx_hbm.at[row], idx_v)
            pltpu.async_copy(table_hbm.at[idx_v], rows_v, sem).wait()
            pltpu.sync_copy(rows_v, out_hbm.at[pl.ds(row * ROW, ROW)])
            return carry

        lax.fori_loop(0, ROWS_PER_W, step, 0)

    return gather_kernel


_gather = _make_gather()


def kernel(dist_mat, table):
    idx = dist_mat.astype(jnp.int32).reshape(NROWS, ROW)
    out = _gather(table, idx)
    return out.reshape(SEQ, SEQ, HIDDEN)


# speedup vs baseline: 3.9868x; 1.1082x over previous
import functools

import jax
import jax.numpy as jnp
from jax import lax
from jax.experimental import pallas as pl
from jax.experimental.pallas import tpu as pltpu
from jax.experimental.pallas import tpu_sc as plsc

SEQ = 2048
HIDDEN = 64
B = SEQ * SEQ
ROW = 128
NROWS = B // ROW
NW = 32
ROWS_PER_W = NROWS // NW
K = 4
NG = ROWS_PER_W // K
NBUF = 2


def _make_gather():
    mesh = plsc.VectorSubcoreMesh(core_axis_name="c", subcore_axis_name="s")

    @functools.partial(
        pl.kernel,
        mesh=mesh,
        out_type=jax.ShapeDtypeStruct((B, HIDDEN), jnp.float32),
        scratch_types=[
            pltpu.VMEM((K, ROW), jnp.int32),
            pltpu.VMEM((K, ROW), jnp.int32),
            pltpu.VMEM((K * ROW, HIDDEN), jnp.float32),
            pltpu.VMEM((K * ROW, HIDDEN), jnp.float32),
            pltpu.SemaphoreType.DMA,
            pltpu.SemaphoreType.DMA,
            pltpu.SemaphoreType.DMA,
            pltpu.SemaphoreType.DMA,
            pltpu.SemaphoreType.DMA,
            pltpu.SemaphoreType.DMA,
        ],
        compiler_params=pltpu.CompilerParams(use_tc_tiling_on_sc=False),
    )
    def gather_kernel(table_hbm, idx_hbm, out_hbm,
                      idx_v0, idx_v1, rows_v0, rows_v1,
                      si0, si1, sg0, sg1, so0, so1):
        idx_bufs = (idx_v0, idx_v1)
        rows_bufs = (rows_v0, rows_v1)
        sem_i = (si0, si1)
        sem_g = (sg0, sg1)
        sem_o = (so0, so1)

        c = lax.axis_index("c")
        s = lax.axis_index("s")
        wid = s * 2 + c
        base_row = wid * ROWS_PER_W

        def idx_start(g, p):
            pltpu.async_copy(
                idx_hbm.at[pl.ds(base_row + g * K, K)], idx_bufs[p], sem_i[p])

        def scatter_desc(g, p):
            return pltpu.make_async_copy(
                rows_bufs[p],
                out_hbm.at[pl.ds((base_row + g * K) * ROW, K * ROW)],
                sem_o[p])

        idx_start(0, 0)

        def group(g, p):
            @pl.when(g >= NBUF)
            def _():
                scatter_desc(g - NBUF, p).wait()

            pltpu.make_async_copy(
                idx_hbm.at[pl.ds(0, K)], idx_bufs[p], sem_i[p]).wait()

            copies = [
                pltpu.async_copy(
                    table_hbm.at[idx_bufs[p].at[j]],
                    rows_bufs[p].at[pl.ds(j * ROW, ROW)],
                    sem_g[p])
                for j in range(K)
            ]

            @pl.when(g + 1 < NG)
            def _():
                idx_start(g + 1, 1 - p)

            for cp in copies:
                cp.wait()

            pltpu.async_copy(
                rows_bufs[p],
                out_hbm.at[pl.ds((base_row + g * K) * ROW, K * ROW)],
                sem_o[p])

        def outer(gg, carry):
            for p in range(NBUF):
                group(gg * NBUF + p, p)
            return carry

        lax.fori_loop(0, NG // NBUF, outer, 0)

        for p in range(NBUF):
            scatter_desc(NG - NBUF + p, p).wait()

    return gather_kernel


_gather = _make_gather()


def kernel(dist_mat, table):
    idx = dist_mat.astype(jnp.int32).reshape(NROWS, ROW)
    out = _gather(table, idx)
    return out.reshape(SEQ, SEQ, HIDDEN)


# speedup vs baseline: 6.1943x; 1.5537x over previous
import functools

import jax
import jax.numpy as jnp
from jax import lax
from jax.experimental import pallas as pl
from jax.experimental.pallas import tpu as pltpu
from jax.experimental.pallas import tpu_sc as plsc

SEQ = 2048
HIDDEN = 64
B = SEQ * SEQ
ROW = 128
NROWS = B // ROW
NW = 32
ROWS_PER_W = NROWS // NW
K = 4
NG = ROWS_PER_W // K
NBUF = 2


def _make_gather():
    mesh = plsc.VectorSubcoreMesh(core_axis_name="c", subcore_axis_name="s")

    @functools.partial(
        pl.kernel,
        mesh=mesh,
        out_type=jax.ShapeDtypeStruct((B, HIDDEN), jnp.float32),
        scratch_types=[
            pltpu.VMEM((K, ROW), jnp.int32),
            pltpu.VMEM((K, ROW), jnp.int32),
            pltpu.VMEM((K * ROW, HIDDEN), jnp.float32),
            pltpu.VMEM((K * ROW, HIDDEN), jnp.float32),
            pltpu.VMEM_SHARED((512, HIDDEN), jnp.float32),
            pltpu.SemaphoreType.DMA,
            pltpu.SemaphoreType.DMA,
            pltpu.SemaphoreType.DMA,
            pltpu.SemaphoreType.DMA,
            pltpu.SemaphoreType.DMA,
            pltpu.SemaphoreType.DMA,
        ],
        compiler_params=pltpu.CompilerParams(use_tc_tiling_on_sc=False),
    )
    def gather_kernel(table_hbm, idx_hbm, out_hbm,
                      idx_v0, idx_v1, rows_v0, rows_v1, table_sp,
                      si0, si1, sg0, sg1, so0, so1):
        idx_bufs = (idx_v0, idx_v1)
        rows_bufs = (rows_v0, rows_v1)
        sem_i = (si0, si1)
        sem_g = (sg0, sg1)
        sem_o = (so0, so1)

        c = lax.axis_index("c")
        s = lax.axis_index("s")
        wid = s * 2 + c
        base_row = wid * ROWS_PER_W

        def idx_start(g, p):
            pltpu.async_copy(
                idx_hbm.at[pl.ds(base_row + g * K, K)], idx_bufs[p], sem_i[p])

        def scatter_desc(g, p):
            return pltpu.make_async_copy(
                rows_bufs[p],
                out_hbm.at[pl.ds((base_row + g * K) * ROW, K * ROW)],
                sem_o[p])

        @pl.when(s == 0)
        def _():
            pltpu.sync_copy(table_hbm, table_sp)

        plsc.subcore_barrier()

        idx_start(0, 0)

        def group(g, p):
            @pl.when(g >= NBUF)
            def _():
                scatter_desc(g - NBUF, p).wait()

            pltpu.make_async_copy(
                idx_hbm.at[pl.ds(0, K)], idx_bufs[p], sem_i[p]).wait()

            copies = [
                pltpu.async_copy(
                    table_sp.at[idx_bufs[p].at[j]],
                    rows_bufs[p].at[pl.ds(j * ROW, ROW)],
                    sem_g[p])
                for j in range(K)
            ]

            @pl.when(g + 1 < NG)
            def _():
                idx_start(g + 1, 1 - p)

            for cp in copies:
                cp.wait()

            pltpu.async_copy(
                rows_bufs[p],
                out_hbm.at[pl.ds((base_row + g * K) * ROW, K * ROW)],
                sem_o[p])

        def outer(gg, carry):
            for p in range(NBUF):
                group(gg * NBUF + p, p)
            return carry

        lax.fori_loop(0, NG // NBUF, outer, 0)

        for p in range(NBUF):
            scatter_desc(NG - NBUF + p, p).wait()

    return gather_kernel


_gather = _make_gather()


def kernel(dist_mat, table):
    idx = dist_mat.astype(jnp.int32).reshape(NROWS, ROW)
    out = _gather(table, idx)
    return out.reshape(SEQ, SEQ, HIDDEN)
